# Initial kernel scaffold; baseline (speedup 1.0000x reference)
import jax
import jax.numpy as jnp
from jax.experimental import pallas as pl


def kernel(x, pe_table):
    raise NotImplementedError("write your pallas kernel here")


# speedup vs baseline: 1.7229x; 1.7229x over previous
import jax
import jax.numpy as jnp
from jax.experimental import pallas as pl

B, T, D = 4, 8192, 1024
BT = 512


def _add_body(x_ref, pe_ref, o_ref):
    o_ref[...] = x_ref[...] + pe_ref[...][None, :, :]


def kernel(x, pe_table):
    grid = (T // BT,)
    return pl.pallas_call(
        _add_body,
        grid=grid,
        in_specs=[
            pl.BlockSpec((B, BT, D), lambda i: (0, i, 0)),
            pl.BlockSpec((BT, D), lambda i: (i, 0)),
        ],
        out_specs=pl.BlockSpec((B, BT, D), lambda i: (0, i, 0)),
        out_shape=jax.ShapeDtypeStruct((B, T, D), jnp.float32),
    )(x, pe_table[:T])
